# baseline (speedup 1.0000x reference)
import functools

import jax
import jax.numpy as jnp
from jax import lax
from jax.experimental import pallas as pl
from jax.experimental.pallas import tpu as pltpu
from jax.experimental.pallas import tpu_sc as plsc

_EPS = 1e-5
_G = 128
_NC = 2
_NS = 16
_CH = 128


def _round_up(a, b):
  return (a + b - 1) // b * b


def _mm0_body(x_ref, w_ref, h_ref):
  h_ref[...] = jnp.dot(x_ref[...], w_ref[...],
                       preferred_element_type=jnp.float32,
                       precision=lax.Precision.HIGHEST)


def _mm1_body(n, h_ref, dp_ref, hp_ref, dinv_ref):
  dp = dp_ref[...]
  deg = 1.0 + dp[0, :, :1] + dp[1, :, :1]
  dinv = lax.rsqrt(deg)
  hp_ref[...] = h_ref[...] * dinv
  dinv_ref[...] = dinv


def _pre_body(accp_ref, hp_ref, dinv_ref, b_ref, pre_ref, s_ref, q_ref):
  ap = accp_ref[...]
  pre = (ap[0] + ap[1] + hp_ref[...]) * dinv_ref[...] + b_ref[...]
  pre_ref[...] = pre
  s = jnp.sum(pre, axis=0, keepdims=True)
  q = jnp.sum(pre * pre, axis=0, keepdims=True)
  i = pl.program_id(0)

  @pl.when(i == 0)
  def _():
    s_ref[...] = s
    q_ref[...] = q

  @pl.when(i != 0)
  def _():
    s_ref[...] += s
    q_ref[...] += q


def _bn_mm2_body(n, pre_ref, s_ref, q_ref, g_ref, be_ref, w_ref, dinv_ref,
                 hp2_ref):
  mu = s_ref[...] * (1.0 / n)
  var = q_ref[...] * (1.0 / n) - mu * mu
  sc = g_ref[...] * lax.rsqrt(var + _EPS)
  r = jnp.maximum((pre_ref[...] - mu) * sc + be_ref[...], 0.0)
  h = jnp.dot(r, w_ref[...], preferred_element_type=jnp.float32,
              precision=lax.Precision.HIGHEST)
  hp2_ref[...] = h * dinv_ref[...]


def _bn_pool_head_body(n, bm, steps, pre_ref, s_ref, q_ref, g_ref, be_ref,
                       batch_ref, w1_ref, b1_ref, w2_ref, b2_ref,
                       ps_ref, pc_ref, out_ref):
  i = pl.program_id(0)

  @pl.when(i < steps)
  def _():
    mu = s_ref[...] * (1.0 / n)
    var = q_ref[...] * (1.0 / n) - mu * mu
    sc = g_ref[...] * lax.rsqrt(var + _EPS)
    r = jnp.maximum((pre_ref[...] - mu) * sc + be_ref[...], 0.0)
    b = batch_ref[...]
    gid = lax.broadcasted_iota(jnp.int32, (bm, _G), 1)
    oh = (b == gid).astype(jnp.float32)
    dn = (((0,), (0,)), ((), ()))
    ps = lax.dot_general(oh, r, dn, preferred_element_type=jnp.float32,
                         precision=lax.Precision.HIGHEST)
    pc = lax.dot_general(oh, jnp.ones((bm, 1), jnp.float32), dn,
                         preferred_element_type=jnp.float32,
                         precision=lax.Precision.HIGHEST)

    @pl.when(i == 0)
    def _():
      ps_ref[...] = ps
      pc_ref[...] = pc

    @pl.when(i != 0)
    def _():
      ps_ref[...] += ps
      pc_ref[...] += pc

  @pl.when(i == steps)
  def _():
    pooled = ps_ref[...] / jnp.maximum(pc_ref[...], 1.0)
    a = jnp.dot(pooled, w1_ref[...], preferred_element_type=jnp.float32,
                precision=lax.Precision.HIGHEST)
    a = jnp.maximum(a + b1_ref[...], 0.0)
    o = jnp.dot(a, w2_ref[...], preferred_element_type=jnp.float32,
                precision=lax.Precision.HIGHEST)
    out_ref[...] = o + b2_ref[...]


@functools.lru_cache(maxsize=None)
def _build(N, E, F, H):
  NW = _NC * _NS
  assert E % NW == 0, E
  EP = E // NW
  NFULL = EP // _CH
  TAIL = EP % _CH
  assert TAIL % 8 == 0 and _CH % 8 == 0
  RT = _round_up(-(-N // _NS), 8)
  NPAD = RT * _NS

  mesh = plsc.VectorSubcoreMesh(core_axis_name="c", subcore_axis_name="s",
                                num_cores=_NC, num_subcores=_NS)

  NBD = 4
  NCHPD = _round_up(-(-EP // _CH), NBD)
  EPPD = NCHPD * _CH
  NGD = NCHPD // NBD
  CHA = _CH
  NB = 3
  NCHP = _round_up(-(-EP // CHA), NB)
  EPP = NCHP * CHA
  NG = NCHP // NB
  deg_scratch = (
      [pltpu.VMEM((NCHPD, _CH), jnp.int32)] +
      [pltpu.VMEM((_CH, 128), jnp.float32)] +
      [pltpu.SemaphoreType.DMA] * NBD +
      [pltpu.VMEM_SHARED((NPAD, 128), jnp.float32)]
  )

  @functools.partial(
      pl.kernel,
      out_type=jax.ShapeDtypeStruct((_NC, NPAD, 128), jnp.float32),
      mesh=mesh,
      scratch_types=deg_scratch,
  )
  def deg_kernel(dstp_hbm, ones_hbm, zeros_hbm, out_hbm, *bufs):
    dv = bufs[0]
    ones_v = bufs[1]
    ss = bufs[2:2 + NBD]
    deg_sh = bufs[2 + NBD]
    cid = lax.axis_index("c")
    sid = lax.axis_index("s")
    wid = cid * _NS + sid
    pltpu.sync_copy(ones_hbm, ones_v)
    pltpu.sync_copy(dstp_hbm.at[wid], dv)
    pltpu.sync_copy(zeros_hbm, deg_sh.at[pl.ds(sid * RT, RT)])
    plsc.subcore_barrier()
    for b in range(NBD):
      pltpu.async_copy(ones_v, deg_sh.at[dv.at[b]], ss[b], add=True)

    def body(j, carry):
      for b in range(NBD):
        c = j * NBD + b
        pltpu.make_async_copy(ones_v, deg_sh.at[dv.at[c]], ss[b]).wait()

        @pl.when(j < NGD - 1)
        def _():
          pltpu.async_copy(ones_v, deg_sh.at[dv.at[c + NBD]], ss[b],
                           add=True)

      return carry

    lax.fori_loop(0, NGD, body, 0)
    plsc.subcore_barrier()
    pltpu.sync_copy(deg_sh.at[pl.ds(sid * RT, RT)],
                    out_hbm.at[cid, pl.ds(sid * RT, RT), :])

  acc_scratch = (
      [pltpu.VMEM((2, CHA), jnp.int32)] * NB +
      [pltpu.VMEM((CHA, F), jnp.float32)] * NB +
      [pltpu.SemaphoreType.DMA] * (2 * NB) +
      [pltpu.VMEM_SHARED((NPAD, F), jnp.float32)]
  )

  @functools.partial(
      pl.kernel,
      out_type=jax.ShapeDtypeStruct((_NC, NPAD, F), jnp.float32),
      mesh=mesh,
      scratch_types=acc_scratch,
  )
  def acc_kernel(hp_hbm, ep_hbm, zrows_hbm, out_hbm, *bufs):
    idx2 = bufs[:NB]
    rows = bufs[NB:2 * NB]
    gs = bufs[2 * NB:3 * NB]
    ss = bufs[3 * NB:4 * NB]
    acc_sh = bufs[4 * NB]
    cid = lax.axis_index("c")
    sid = lax.axis_index("s")
    wid = cid * _NS + sid
    pltpu.sync_copy(zrows_hbm, acc_sh.at[pl.ds(sid * RT, RT)])
    plsc.subcore_barrier()
    base = wid * NCHP
    for b in range(NB):
      pltpu.sync_copy(ep_hbm.at[base + b], idx2[b])
      pltpu.async_copy(hp_hbm.at[idx2[b].at[0]], rows[b], gs[b])

    def body(j, carry):
      for b in range(NB):
        c = j * NB + b
        pltpu.make_async_copy(hp_hbm.at[idx2[b].at[0]], rows[b], gs[b]).wait()
        pltpu.async_copy(rows[b], acc_sh.at[idx2[b].at[1]], ss[b], add=True)

        @pl.when(j < NG - 1)
        def _():
          pltpu.make_async_copy(rows[b], acc_sh.at[idx2[b].at[1]],
                                ss[b]).wait()
          pltpu.sync_copy(ep_hbm.at[base + c + NB], idx2[b])
          pltpu.async_copy(hp_hbm.at[idx2[b].at[0]], rows[b], gs[b])

      return carry

    lax.fori_loop(0, NG, body, 0)
    for b in range(NB):
      pltpu.make_async_copy(rows[b], acc_sh.at[idx2[b].at[1]], ss[b]).wait()
    plsc.subcore_barrier()
    pltpu.sync_copy(acc_sh.at[pl.ds(sid * RT, RT)],
                    out_hbm.at[cid, pl.ds(sid * RT, RT), :])

  bm = max(d for d in range(8, min(N, 2048) + 1, 8) if N % d == 0)
  steps = N // bm
  f32 = jnp.float32

  mm0 = pl.pallas_call(
      _mm0_body,
      grid=(steps,),
      in_specs=[
          pl.BlockSpec((bm, F), lambda i: (i, 0)),
          pl.BlockSpec((F, H), lambda i: (0, 0)),
      ],
      out_specs=pl.BlockSpec((bm, H), lambda i: (i, 0)),
      out_shape=jax.ShapeDtypeStruct((N, H), f32),
  )

  mm1 = pl.pallas_call(
      functools.partial(_mm1_body, N),
      grid=(steps,),
      in_specs=[
          pl.BlockSpec((bm, H), lambda i: (i, 0)),
          pl.BlockSpec((_NC, bm, 128), lambda i: (0, i, 0)),
      ],
      out_specs=[
          pl.BlockSpec((bm, H), lambda i: (i, 0)),
          pl.BlockSpec((bm, 1), lambda i: (i, 0)),
      ],
      out_shape=[
          jax.ShapeDtypeStruct((N, H), f32),
          jax.ShapeDtypeStruct((N, 1), f32),
      ],
  )

  pre = pl.pallas_call(
      _pre_body,
      grid=(steps,),
      in_specs=[
          pl.BlockSpec((_NC, bm, H), lambda i: (0, i, 0)),
          pl.BlockSpec((bm, H), lambda i: (i, 0)),
          pl.BlockSpec((bm, 1), lambda i: (i, 0)),
          pl.BlockSpec((1, H), lambda i: (0, 0)),
      ],
      out_specs=[
          pl.BlockSpec((bm, H), lambda i: (i, 0)),
          pl.BlockSpec((1, H), lambda i: (0, 0)),
          pl.BlockSpec((1, H), lambda i: (0, 0)),
      ],
      out_shape=[
          jax.ShapeDtypeStruct((N, H), f32),
          jax.ShapeDtypeStruct((1, H), f32),
          jax.ShapeDtypeStruct((1, H), f32),
      ],
  )

  bn_mm2 = pl.pallas_call(
      functools.partial(_bn_mm2_body, N),
      grid=(steps,),
      in_specs=[
          pl.BlockSpec((bm, H), lambda i: (i, 0)),
          pl.BlockSpec((1, H), lambda i: (0, 0)),
          pl.BlockSpec((1, H), lambda i: (0, 0)),
          pl.BlockSpec((1, H), lambda i: (0, 0)),
          pl.BlockSpec((1, H), lambda i: (0, 0)),
          pl.BlockSpec((H, H), lambda i: (0, 0)),
          pl.BlockSpec((bm, 1), lambda i: (i, 0)),
      ],
      out_specs=pl.BlockSpec((bm, H), lambda i: (i, 0)),
      out_shape=jax.ShapeDtypeStruct((N, H), f32),
  )

  clamp = lambda i: (min(i, steps - 1) if isinstance(i, int)
                     else jnp.minimum(i, steps - 1), 0)
  bn_pool_head = pl.pallas_call(
      functools.partial(_bn_pool_head_body, N, bm, steps),
      grid=(steps + 1,),
      in_specs=[
          pl.BlockSpec((bm, H), clamp),
          pl.BlockSpec((1, H), lambda i: (0, 0)),
          pl.BlockSpec((1, H), lambda i: (0, 0)),
          pl.BlockSpec((1, H), lambda i: (0, 0)),
          pl.BlockSpec((1, H), lambda i: (0, 0)),
          pl.BlockSpec((bm, 1), clamp),
          pl.BlockSpec((H, 32), lambda i: (0, 0)),
          pl.BlockSpec((1, 32), lambda i: (0, 0)),
          pl.BlockSpec((32, 2), lambda i: (0, 0)),
          pl.BlockSpec((1, 2), lambda i: (0, 0)),
      ],
      out_specs=[
          pl.BlockSpec((_G, H), lambda i: (0, 0)),
          pl.BlockSpec((_G, 1), lambda i: (0, 0)),
          pl.BlockSpec((_G, 2), lambda i: (0, 0)),
      ],
      out_shape=[
          jax.ShapeDtypeStruct((_G, H), f32),
          jax.ShapeDtypeStruct((_G, 1), f32),
          jax.ShapeDtypeStruct((_G, 2), f32),
      ],
  )

  return dict(deg=deg_kernel, acc=acc_kernel, mm0=mm0, mm1=mm1, pre=pre,
              bn_mm2=bn_mm2, bn_pool_head=bn_pool_head,
              RT=RT, NPAD=NPAD, EP=EP, EPP=EPP, EPPD=EPPD)


def kernel(x, edge_index, batch, W1, b1, gamma1, beta1, W2, b2, gamma2,
           beta2, fc1_W, fc1_b, fc2_W, fc2_b):
  N, F = x.shape
  H = W1.shape[1]
  E = edge_index.shape[1]
  k = _build(N, E, F, H)
  RT = k["RT"]

  NW = _NC * _NS
  EP = k["EP"]
  NPAD = k["NPAD"]
  EPP = k["EPP"]
  EPPD = k["EPPD"]
  src = edge_index[0]
  dst = edge_index[1]
  srcr = src.reshape(NW, EP)
  dstr = dst.reshape(NW, EP)
  tile = jnp.arange(NW, dtype=jnp.int32)[:, None]

  def padded(base2d, epp, trash):
    ne = epp - EP
    lane = jnp.arange(ne, dtype=jnp.int32)[None, :]
    if trash:
      p = N + (tile * ne + lane) % (NPAD - N)
    else:
      p = (tile * ne + lane) % N
    return jnp.concatenate([base2d, p], axis=1).reshape(NW * epp)

  NCHP = EPP // _CH
  sp3 = padded(srcr, EPP, False).reshape(NW, NCHP, _CH)
  dp3 = padded(dstr, EPP, True).reshape(NW, NCHP, _CH)
  epair = jnp.stack([sp3, dp3], axis=2).reshape(NW * NCHP, 2, _CH)
  dstpd = padded(dstr, EPPD, True).reshape(NW, EPPD // _CH, _CH)
  f32 = jnp.float32
  zrows = jnp.zeros((RT, F), f32)
  ones_ch = jnp.ones((_CH, 128), f32)
  zeros_rt = jnp.zeros((RT, 128), f32)

  h1 = k["mm0"](x, W1)
  degp = k["deg"](dstpd, ones_ch, zeros_rt)
  hp1, dinv = k["mm1"](h1, degp)

  accp1 = k["acc"](hp1, epair, zrows)
  pre1, s1, q1 = k["pre"](accp1, hp1, dinv, b1.reshape(1, H))
  hp2 = k["bn_mm2"](pre1, s1, q1, gamma1.reshape(1, H), beta1.reshape(1, H),
                    W2, dinv)

  accp2 = k["acc"](hp2, epair, zrows)
  pre2, s2, q2 = k["pre"](accp2, hp2, dinv, b2.reshape(1, H))
  ps, pc, out = k["bn_pool_head"](
      pre2, s2, q2, gamma2.reshape(1, H), beta2.reshape(1, H),
      batch.reshape(N, 1), fc1_W, fc1_b.reshape(1, -1),
      fc2_W, fc2_b.reshape(1, -1))
  return out


# speedup vs baseline: 1.0034x; 1.0034x over previous
import functools

import jax
import jax.numpy as jnp
from jax import lax
from jax.experimental import pallas as pl
from jax.experimental.pallas import tpu as pltpu
from jax.experimental.pallas import tpu_sc as plsc

_EPS = 1e-5
_G = 128
_NC = 2
_NS = 16
_CH = 128


def _round_up(a, b):
  return (a + b - 1) // b * b


def _mm1_body(n, x_ref, w_ref, dp_ref, hp_ref, dinv_ref):
  dp = dp_ref[...]
  deg = 1.0 + dp[0, :, :1] + dp[1, :, :1]
  dinv = lax.rsqrt(deg)
  h = jnp.dot(x_ref[...], w_ref[...], preferred_element_type=jnp.float32,
              precision=lax.Precision.HIGHEST)
  hp_ref[...] = h * dinv
  dinv_ref[...] = dinv


def _pre_block(accp_ref, hp_ref, dinv_ref, b_ref):
  ap = accp_ref[...]
  return (ap[0] + ap[1] + hp_ref[...]) * dinv_ref[...] + b_ref[...]


def _stats_mm2_body(n, steps, accp_ref, hp_ref, dinv_ref, b_ref, g_ref,
                    be_ref, w_ref, hp2_ref, s_ref, q_ref):
  i = pl.program_id(0)
  pre = _pre_block(accp_ref, hp_ref, dinv_ref, b_ref)

  @pl.when(i == 0)
  def _():
    s_ref[...] = jnp.zeros_like(s_ref)
    q_ref[...] = jnp.zeros_like(q_ref)

  @pl.when(i < steps)
  def _():
    s_ref[...] += jnp.sum(pre, axis=0, keepdims=True)
    q_ref[...] += jnp.sum(pre * pre, axis=0, keepdims=True)

  @pl.when(i >= steps)
  def _():
    mu = s_ref[...] * (1.0 / n)
    var = q_ref[...] * (1.0 / n) - mu * mu
    sc = g_ref[...] * lax.rsqrt(var + _EPS)
    r = jnp.maximum((pre - mu) * sc + be_ref[...], 0.0)
    h = jnp.dot(r, w_ref[...], preferred_element_type=jnp.float32,
                precision=lax.Precision.HIGHEST)
    hp2_ref[...] = h * dinv_ref[...]


def _stats_pool_head_body(n, bm, steps, accp_ref, hp_ref, dinv_ref, b_ref,
                          g_ref, be_ref, batch_ref, w1_ref, b1_ref, w2_ref,
                          b2_ref, ps_ref, pc_ref, out_ref, s_ref, q_ref):
  i = pl.program_id(0)

  @pl.when(i == 0)
  def _():
    s_ref[...] = jnp.zeros_like(s_ref)
    q_ref[...] = jnp.zeros_like(q_ref)

  @pl.when(i < 2 * steps)
  def _():
    pre = _pre_block(accp_ref, hp_ref, dinv_ref, b_ref)

    @pl.when(i < steps)
    def _():
      s_ref[...] += jnp.sum(pre, axis=0, keepdims=True)
      q_ref[...] += jnp.sum(pre * pre, axis=0, keepdims=True)

    @pl.when(i >= steps)
    def _():
      mu = s_ref[...] * (1.0 / n)
      var = q_ref[...] * (1.0 / n) - mu * mu
      sc = g_ref[...] * lax.rsqrt(var + _EPS)
      r = jnp.maximum((pre - mu) * sc + be_ref[...], 0.0)
      bt = batch_ref[...]
      gid = lax.broadcasted_iota(jnp.int32, (bm, _G), 1)
      oh = (bt == gid).astype(jnp.float32)
      dn = (((0,), (0,)), ((), ()))
      ps = lax.dot_general(oh, r, dn, preferred_element_type=jnp.float32,
                           precision=lax.Precision.HIGHEST)
      pc = lax.dot_general(oh, jnp.ones((bm, 1), jnp.float32), dn,
                           preferred_element_type=jnp.float32,
                           precision=lax.Precision.HIGHEST)

      @pl.when(i == steps)
      def _():
        ps_ref[...] = ps
        pc_ref[...] = pc

      @pl.when(i != steps)
      def _():
        ps_ref[...] += ps
        pc_ref[...] += pc

  @pl.when(i == 2 * steps)
  def _():
    pooled = ps_ref[...] / jnp.maximum(pc_ref[...], 1.0)
    a = jnp.dot(pooled, w1_ref[...], preferred_element_type=jnp.float32,
                precision=lax.Precision.HIGHEST)
    a = jnp.maximum(a + b1_ref[...], 0.0)
    o = jnp.dot(a, w2_ref[...], preferred_element_type=jnp.float32,
                precision=lax.Precision.HIGHEST)
    out_ref[...] = o + b2_ref[...]


def _build(N, E, F, H):
  NW = _NC * _NS
  assert E % NW == 0, E
  EP = E // NW
  NFULL = EP // _CH
  TAIL = EP % _CH
  assert TAIL % 8 == 0 and _CH % 8 == 0
  RT = _round_up(-(-N // _NS), 8)
  NPAD = RT * _NS

  mesh = plsc.VectorSubcoreMesh(core_axis_name="c", subcore_axis_name="s",
                                num_cores=_NC, num_subcores=_NS)

  NBD = 4
  NCHPD = _round_up(-(-EP // _CH), NBD)
  EPPD = NCHPD * _CH
  NGD = NCHPD // NBD
  CHA = _CH
  NB = 3
  NCHP = _round_up(-(-EP // CHA), NB)
  EPP = NCHP * CHA
  NG = NCHP // NB
  deg_scratch = (
      [pltpu.VMEM((NCHPD, _CH), jnp.int32)] +
      [pltpu.VMEM((_CH, 128), jnp.float32)] +
      [pltpu.SemaphoreType.DMA] * NBD +
      [pltpu.VMEM_SHARED((NPAD, 128), jnp.float32)]
  )

  @functools.partial(
      pl.kernel,
      out_type=jax.ShapeDtypeStruct((_NC, NPAD, 128), jnp.float32),
      mesh=mesh,
      scratch_types=deg_scratch,
  )
  def deg_kernel(dstp_hbm, ones_hbm, zeros_hbm, out_hbm, *bufs):
    dv = bufs[0]
    ones_v = bufs[1]
    ss = bufs[2:2 + NBD]
    deg_sh = bufs[2 + NBD]
    cid = lax.axis_index("c")
    sid = lax.axis_index("s")
    wid = cid * _NS + sid
    pltpu.sync_copy(ones_hbm, ones_v)
    pltpu.sync_copy(dstp_hbm.at[wid], dv)
    pltpu.sync_copy(zeros_hbm, deg_sh.at[pl.ds(sid * RT, RT)])
    plsc.subcore_barrier()
    for b in range(NBD):
      pltpu.async_copy(ones_v, deg_sh.at[dv.at[b]], ss[b], add=True)

    def body(j, carry):
      for b in range(NBD):
        c = j * NBD + b
        pltpu.make_async_copy(ones_v, deg_sh.at[dv.at[c]], ss[b]).wait()

        @pl.when(j < NGD - 1)
        def _():
          pltpu.async_copy(ones_v, deg_sh.at[dv.at[c + NBD]], ss[b],
                           add=True)

      return carry

    lax.fori_loop(0, NGD, body, 0)
    plsc.subcore_barrier()
    pltpu.sync_copy(deg_sh.at[pl.ds(sid * RT, RT)],
                    out_hbm.at[cid, pl.ds(sid * RT, RT), :])

  acc_scratch = (
      [pltpu.VMEM((2, CHA), jnp.int32)] * NB +
      [pltpu.VMEM((CHA, F), jnp.float32)] * NB +
      [pltpu.SemaphoreType.DMA] * (2 * NB) +
      [pltpu.VMEM_SHARED((NPAD, F), jnp.float32)]
  )

  @functools.partial(
      pl.kernel,
      out_type=jax.ShapeDtypeStruct((_NC, NPAD, F), jnp.float32),
      mesh=mesh,
      scratch_types=acc_scratch,
  )
  def acc_kernel(hp_hbm, ep_hbm, zrows_hbm, out_hbm, *bufs):
    idx2 = bufs[:NB]
    rows = bufs[NB:2 * NB]
    gs = bufs[2 * NB:3 * NB]
    ss = bufs[3 * NB:4 * NB]
    acc_sh = bufs[4 * NB]
    cid = lax.axis_index("c")
    sid = lax.axis_index("s")
    wid = cid * _NS + sid
    pltpu.sync_copy(zrows_hbm, acc_sh.at[pl.ds(sid * RT, RT)])
    plsc.subcore_barrier()
    base = wid * NCHP
    for b in range(NB):
      pltpu.sync_copy(ep_hbm.at[base + b], idx2[b])
      pltpu.async_copy(hp_hbm.at[idx2[b].at[0]], rows[b], gs[b])

    def body(j, carry):
      for b in range(NB):
        c = j * NB + b
        pltpu.make_async_copy(hp_hbm.at[idx2[b].at[0]], rows[b], gs[b]).wait()
        pltpu.async_copy(rows[b], acc_sh.at[idx2[b].at[1]], ss[b], add=True)

        @pl.when(j < NG - 1)
        def _():
          pltpu.make_async_copy(rows[b], acc_sh.at[idx2[b].at[1]],
                                ss[b]).wait()
          pltpu.sync_copy(ep_hbm.at[base + c + NB], idx2[b])
          pltpu.async_copy(hp_hbm.at[idx2[b].at[0]], rows[b], gs[b])

      return carry

    lax.fori_loop(0, NG, body, 0)
    for b in range(NB):
      pltpu.make_async_copy(rows[b], acc_sh.at[idx2[b].at[1]], ss[b]).wait()
    plsc.subcore_barrier()
    pltpu.sync_copy(acc_sh.at[pl.ds(sid * RT, RT)],
                    out_hbm.at[cid, pl.ds(sid * RT, RT), :])

  bm = max(d for d in range(8, min(N, 2048) + 1, 8) if N % d == 0)
  steps = N // bm
  f32 = jnp.float32

  mm1 = pl.pallas_call(
      functools.partial(_mm1_body, N),
      grid=(steps,),
      in_specs=[
          pl.BlockSpec((bm, F), lambda i: (i, 0)),
          pl.BlockSpec((F, H), lambda i: (0, 0)),
          pl.BlockSpec((_NC, bm, 128), lambda i: (0, i, 0)),
      ],
      out_specs=[
          pl.BlockSpec((bm, H), lambda i: (i, 0)),
          pl.BlockSpec((bm, 1), lambda i: (i, 0)),
      ],
      out_shape=[
          jax.ShapeDtypeStruct((N, H), f32),
          jax.ShapeDtypeStruct((N, 1), f32),
      ],
  )

  wrap = lambda i: (jnp.where(i < steps, i, jnp.minimum(i - steps, steps - 1)), 0)
  wrap3 = lambda i: (0, jnp.where(i < steps, i, jnp.minimum(i - steps, steps - 1)), 0)
  const = lambda i: (0, 0)

  stats_mm2 = pl.pallas_call(
      functools.partial(_stats_mm2_body, N, steps),
      grid=(2 * steps,),
      in_specs=[
          pl.BlockSpec((_NC, bm, H), wrap3),
          pl.BlockSpec((bm, H), wrap),
          pl.BlockSpec((bm, 1), wrap),
          pl.BlockSpec((1, H), const),
          pl.BlockSpec((1, H), const),
          pl.BlockSpec((1, H), const),
          pl.BlockSpec((H, H), const),
      ],
      out_specs=[
          pl.BlockSpec((bm, H), wrap),
          pl.BlockSpec((1, H), const),
          pl.BlockSpec((1, H), const),
      ],
      out_shape=[
          jax.ShapeDtypeStruct((N, H), f32),
          jax.ShapeDtypeStruct((1, H), f32),
          jax.ShapeDtypeStruct((1, H), f32),
      ],
  )

  stats_pool_head = pl.pallas_call(
      functools.partial(_stats_pool_head_body, N, bm, steps),
      grid=(2 * steps + 1,),
      in_specs=[
          pl.BlockSpec((_NC, bm, H), wrap3),
          pl.BlockSpec((bm, H), wrap),
          pl.BlockSpec((bm, 1), wrap),
          pl.BlockSpec((1, H), const),
          pl.BlockSpec((1, H), const),
          pl.BlockSpec((1, H), const),
          pl.BlockSpec((bm, 1), wrap),
          pl.BlockSpec((H, 32), const),
          pl.BlockSpec((1, 32), const),
          pl.BlockSpec((32, 2), const),
          pl.BlockSpec((1, 2), const),
      ],
      out_specs=[
          pl.BlockSpec((_G, H), const),
          pl.BlockSpec((_G, 1), const),
          pl.BlockSpec((_G, 2), const),
          pl.BlockSpec((1, H), const),
          pl.BlockSpec((1, H), const),
      ],
      out_shape=[
          jax.ShapeDtypeStruct((_G, H), f32),
          jax.ShapeDtypeStruct((_G, 1), f32),
          jax.ShapeDtypeStruct((_G, 2), f32),
          jax.ShapeDtypeStruct((1, H), f32),
          jax.ShapeDtypeStruct((1, H), f32),
      ],
  )

  return dict(deg=deg_kernel, acc=acc_kernel, mm1=mm1,
              stats_mm2=stats_mm2, stats_pool_head=stats_pool_head,
              RT=RT, NPAD=NPAD, EP=EP, EPP=EPP, EPPD=EPPD)


def kernel(x, edge_index, batch, W1, b1, gamma1, beta1, W2, b2, gamma2,
           beta2, fc1_W, fc1_b, fc2_W, fc2_b):
  N, F = x.shape
  H = W1.shape[1]
  E = edge_index.shape[1]
  k = _build(N, E, F, H)
  RT = k["RT"]

  NW = _NC * _NS
  EP = k["EP"]
  NPAD = k["NPAD"]
  EPP = k["EPP"]
  EPPD = k["EPPD"]
  src = edge_index[0]
  dst = edge_index[1]
  srcr = src.reshape(NW, EP)
  dstr = dst.reshape(NW, EP)
  tile = jnp.arange(NW, dtype=jnp.int32)[:, None]

  def padded(base2d, epp, trash):
    ne = epp - EP
    lane = jnp.arange(ne, dtype=jnp.int32)[None, :]
    if trash:
      p = N + (tile * ne + lane) % (NPAD - N)
    else:
      p = (tile * ne + lane) % N
    return jnp.concatenate([base2d, p], axis=1).reshape(NW * epp)

  NCHP = EPP // _CH
  sp3 = padded(srcr, EPP, False).reshape(NW, NCHP, _CH)
  dp3 = padded(dstr, EPP, True).reshape(NW, NCHP, _CH)
  epair = jnp.stack([sp3, dp3], axis=2).reshape(NW * NCHP, 2, _CH)
  dstpd = padded(dstr, EPPD, True).reshape(NW, EPPD // _CH, _CH)
  f32 = jnp.float32
  zrows = jnp.zeros((RT, F), f32)
  ones_ch = jnp.ones((_CH, 128), f32)
  zeros_rt = jnp.zeros((RT, 128), f32)

  degp = k["deg"](dstpd, ones_ch, zeros_rt)
  hp1, dinv = k["mm1"](x, W1, degp)

  accp1 = k["acc"](hp1, epair, zrows)
  hp2, _, _ = k["stats_mm2"](accp1, hp1, dinv, b1.reshape(1, H),
                             gamma1.reshape(1, H), beta1.reshape(1, H), W2)

  accp2 = k["acc"](hp2, epair, zrows)
  ps, pc, out, _, _ = k["stats_pool_head"](
      accp2, hp2, dinv, b2.reshape(1, H), gamma2.reshape(1, H),
      beta2.reshape(1, H), batch.reshape(N, 1), fc1_W, fc1_b.reshape(1, -1),
      fc2_W, fc2_b.reshape(1, -1))
  return out


# speedup vs baseline: 1.0038x; 1.0004x over previous
import functools

import jax
import jax.numpy as jnp
from jax import lax
from jax.experimental import pallas as pl
from jax.experimental.pallas import tpu as pltpu
from jax.experimental.pallas import tpu_sc as plsc

_EPS = 1e-5
_G = 128
_NC = 2
_NS = 16
_CH = 128


def _round_up(a, b):
  return (a + b - 1) // b * b


def _mm1_body(n, x_ref, w_ref, dp_ref, hp_ref, dinv_ref):
  dp = dp_ref[...]
  deg = 1.0 + dp[0, :, :1] + dp[1, :, :1]
  dinv = lax.rsqrt(deg)
  h = jnp.dot(x_ref[...], w_ref[...], preferred_element_type=jnp.float32,
              precision=lax.Precision.HIGHEST)
  hp_ref[...] = h * dinv
  dinv_ref[...] = dinv


def _pre_block(accp_ref, hp_ref, dinv_ref, b_ref):
  ap = accp_ref[...]
  return (ap[0] + ap[1] + hp_ref[...]) * dinv_ref[...] + b_ref[...]


def _stats_mm2_body(n, steps, accp_ref, hp_ref, dinv_ref, b_ref, g_ref,
                    be_ref, w_ref, hp2_ref, s_ref, q_ref):
  i = pl.program_id(0)
  pre = _pre_block(accp_ref, hp_ref, dinv_ref, b_ref)

  @pl.when(i == 0)
  def _():
    s_ref[...] = jnp.zeros_like(s_ref)
    q_ref[...] = jnp.zeros_like(q_ref)

  @pl.when(i < steps)
  def _():
    s_ref[...] += jnp.sum(pre, axis=0, keepdims=True)
    q_ref[...] += jnp.sum(pre * pre, axis=0, keepdims=True)

  @pl.when(i >= steps)
  def _():
    mu = s_ref[...] * (1.0 / n)
    var = q_ref[...] * (1.0 / n) - mu * mu
    sc = g_ref[...] * lax.rsqrt(var + _EPS)
    r = jnp.maximum((pre - mu) * sc + be_ref[...], 0.0)
    h = jnp.dot(r, w_ref[...], preferred_element_type=jnp.float32,
                precision=lax.Precision.HIGHEST)
    hp2_ref[...] = h * dinv_ref[...]


def _stats_pool_head_body(n, bm, steps, accp_ref, hp_ref, dinv_ref, b_ref,
                          g_ref, be_ref, batch_ref, w1_ref, b1_ref, w2_ref,
                          b2_ref, ps_ref, pc_ref, out_ref, s_ref, q_ref):
  i = pl.program_id(0)

  @pl.when(i == 0)
  def _():
    s_ref[...] = jnp.zeros_like(s_ref)
    q_ref[...] = jnp.zeros_like(q_ref)

  @pl.when(i < 2 * steps)
  def _():
    pre = _pre_block(accp_ref, hp_ref, dinv_ref, b_ref)

    @pl.when(i < steps)
    def _():
      s_ref[...] += jnp.sum(pre, axis=0, keepdims=True)
      q_ref[...] += jnp.sum(pre * pre, axis=0, keepdims=True)

    @pl.when(i >= steps)
    def _():
      mu = s_ref[...] * (1.0 / n)
      var = q_ref[...] * (1.0 / n) - mu * mu
      sc = g_ref[...] * lax.rsqrt(var + _EPS)
      r = jnp.maximum((pre - mu) * sc + be_ref[...], 0.0)
      bt = batch_ref[...]
      gid = lax.broadcasted_iota(jnp.int32, (bm, _G), 1)
      oh = (bt == gid).astype(jnp.float32)
      dn = (((0,), (0,)), ((), ()))
      ps = lax.dot_general(oh, r, dn, preferred_element_type=jnp.float32,
                           precision=lax.Precision.HIGHEST)
      pc = lax.dot_general(oh, jnp.ones((bm, 1), jnp.float32), dn,
                           preferred_element_type=jnp.float32,
                           precision=lax.Precision.HIGHEST)

      @pl.when(i == steps)
      def _():
        ps_ref[...] = ps
        pc_ref[...] = pc

      @pl.when(i != steps)
      def _():
        ps_ref[...] += ps
        pc_ref[...] += pc

  @pl.when(i == 2 * steps)
  def _():
    pooled = ps_ref[...] / jnp.maximum(pc_ref[...], 1.0)
    a = jnp.dot(pooled, w1_ref[...], preferred_element_type=jnp.float32,
                precision=lax.Precision.HIGHEST)
    a = jnp.maximum(a + b1_ref[...], 0.0)
    o = jnp.dot(a, w2_ref[...], preferred_element_type=jnp.float32,
                precision=lax.Precision.HIGHEST)
    out_ref[...] = o + b2_ref[...]


@functools.lru_cache(maxsize=None)
def _build(N, E, F, H):
  NW = _NC * _NS
  assert E % NW == 0, E
  EP = E // NW
  RT = _round_up(-(-N // _NS), 8)
  NPAD = RT * _NS

  mesh = plsc.VectorSubcoreMesh(core_axis_name="c", subcore_axis_name="s",
                                num_cores=_NC, num_subcores=_NS)

  NBD = 4
  NCHPD = _round_up(-(-EP // _CH), NBD)
  EPPD = NCHPD * _CH
  NGD = NCHPD // NBD
  CHA = _CH
  NB = 3
  NCHP = _round_up(-(-EP // CHA), NB)
  EPP = NCHP * CHA
  NG = NCHP // NB
  deg_scratch = (
      [pltpu.VMEM((NCHPD, _CH), jnp.int32)] +
      [pltpu.VMEM((_CH, 128), jnp.float32)] +
      [pltpu.SemaphoreType.DMA] * NBD +
      [pltpu.VMEM_SHARED((NPAD, 128), jnp.float32)]
  )

  @functools.partial(
      pl.kernel,
      out_type=jax.ShapeDtypeStruct((_NC, NPAD, 128), jnp.float32),
      mesh=mesh,
      scratch_types=deg_scratch,
  )
  def deg_kernel(dstp_hbm, ones_hbm, zeros_hbm, out_hbm, *bufs):
    dv = bufs[0]
    ones_v = bufs[1]
    ss = bufs[2:2 + NBD]
    deg_sh = bufs[2 + NBD]
    cid = lax.axis_index("c")
    sid = lax.axis_index("s")
    wid = cid * _NS + sid
    pltpu.sync_copy(ones_hbm, ones_v)
    pltpu.sync_copy(dstp_hbm.at[wid], dv)
    pltpu.sync_copy(zeros_hbm, deg_sh.at[pl.ds(sid * RT, RT)])
    plsc.subcore_barrier()
    for b in range(NBD):
      pltpu.async_copy(ones_v, deg_sh.at[dv.at[b]], ss[b], add=True)

    def body(j, carry):
      for b in range(NBD):
        c = j * NBD + b
        pltpu.make_async_copy(ones_v, deg_sh.at[dv.at[c]], ss[b]).wait()

        @pl.when(j < NGD - 1)
        def _():
          pltpu.async_copy(ones_v, deg_sh.at[dv.at[c + NBD]], ss[b],
                           add=True)

      return carry

    lax.fori_loop(0, NGD, body, 0)
    plsc.subcore_barrier()
    pltpu.sync_copy(deg_sh.at[pl.ds(sid * RT, RT)],
                    out_hbm.at[cid, pl.ds(sid * RT, RT), :])

  acc_scratch = (
      [pltpu.VMEM((2, CHA), jnp.int32)] * NB +
      [pltpu.VMEM((CHA, F), jnp.float32)] * NB +
      [pltpu.SemaphoreType.DMA] * (2 * NB) +
      [pltpu.VMEM_SHARED((NPAD, F), jnp.float32)]
  )

  @functools.partial(
      pl.kernel,
      out_type=jax.ShapeDtypeStruct((_NC, NPAD, F), jnp.float32),
      mesh=mesh,
      scratch_types=acc_scratch,
  )
  def acc_kernel(hp_hbm, ep_hbm, zrows_hbm, out_hbm, *bufs):
    idx2 = bufs[:NB]
    rows = bufs[NB:2 * NB]
    gs = bufs[2 * NB:3 * NB]
    ss = bufs[3 * NB:4 * NB]
    acc_sh = bufs[4 * NB]
    cid = lax.axis_index("c")
    sid = lax.axis_index("s")
    wid = cid * _NS + sid
    pltpu.sync_copy(zrows_hbm, acc_sh.at[pl.ds(sid * RT, RT)])
    plsc.subcore_barrier()
    base = wid * NCHP
    for b in range(NB):
      pltpu.sync_copy(ep_hbm.at[base + b], idx2[b])
      pltpu.async_copy(hp_hbm.at[idx2[b].at[0]], rows[b], gs[b])

    def body(j, carry):
      for b in range(NB):
        c = j * NB + b
        pltpu.make_async_copy(hp_hbm.at[idx2[b].at[0]], rows[b], gs[b]).wait()
        pltpu.async_copy(rows[b], acc_sh.at[idx2[b].at[1]], ss[b], add=True)

        @pl.when(j < NG - 1)
        def _():
          pltpu.make_async_copy(rows[b], acc_sh.at[idx2[b].at[1]],
                                ss[b]).wait()
          pltpu.sync_copy(ep_hbm.at[base + c + NB], idx2[b])
          pltpu.async_copy(hp_hbm.at[idx2[b].at[0]], rows[b], gs[b])

      return carry

    lax.fori_loop(0, NG, body, 0)
    for b in range(NB):
      pltpu.make_async_copy(rows[b], acc_sh.at[idx2[b].at[1]], ss[b]).wait()
    plsc.subcore_barrier()
    pltpu.sync_copy(acc_sh.at[pl.ds(sid * RT, RT)],
                    out_hbm.at[cid, pl.ds(sid * RT, RT), :])

  bm = max(d for d in range(8, min(N, 2048) + 1, 8) if N % d == 0)
  steps = N // bm
  f32 = jnp.float32

  mm1 = pl.pallas_call(
      functools.partial(_mm1_body, N),
      grid=(steps,),
      in_specs=[
          pl.BlockSpec((bm, F), lambda i: (i, 0)),
          pl.BlockSpec((F, H), lambda i: (0, 0)),
          pl.BlockSpec((_NC, bm, 128), lambda i: (0, i, 0)),
      ],
      out_specs=[
          pl.BlockSpec((bm, H), lambda i: (i, 0)),
          pl.BlockSpec((bm, 1), lambda i: (i, 0)),
      ],
      out_shape=[
          jax.ShapeDtypeStruct((N, H), f32),
          jax.ShapeDtypeStruct((N, 1), f32),
      ],
  )

  wrap = lambda i: (jnp.where(i < steps, i, jnp.minimum(i - steps, steps - 1)), 0)
  wrap3 = lambda i: (0, jnp.where(i < steps, i, jnp.minimum(i - steps, steps - 1)), 0)
  const = lambda i: (0, 0)

  stats_mm2 = pl.pallas_call(
      functools.partial(_stats_mm2_body, N, steps),
      grid=(2 * steps,),
      in_specs=[
          pl.BlockSpec((_NC, bm, H), wrap3),
          pl.BlockSpec((bm, H), wrap),
          pl.BlockSpec((bm, 1), wrap),
          pl.BlockSpec((1, H), const),
          pl.BlockSpec((1, H), const),
          pl.BlockSpec((1, H), const),
          pl.BlockSpec((H, H), const),
      ],
      out_specs=[
          pl.BlockSpec((bm, H), wrap),
          pl.BlockSpec((1, H), const),
          pl.BlockSpec((1, H), const),
      ],
      out_shape=[
          jax.ShapeDtypeStruct((N, H), f32),
          jax.ShapeDtypeStruct((1, H), f32),
          jax.ShapeDtypeStruct((1, H), f32),
      ],
  )

  stats_pool_head = pl.pallas_call(
      functools.partial(_stats_pool_head_body, N, bm, steps),
      grid=(2 * steps + 1,),
      in_specs=[
          pl.BlockSpec((_NC, bm, H), wrap3),
          pl.BlockSpec((bm, H), wrap),
          pl.BlockSpec((bm, 1), wrap),
          pl.BlockSpec((1, H), const),
          pl.BlockSpec((1, H), const),
          pl.BlockSpec((1, H), const),
          pl.BlockSpec((bm, 1), wrap),
          pl.BlockSpec((H, 32), const),
          pl.BlockSpec((1, 32), const),
          pl.BlockSpec((32, 2), const),
          pl.BlockSpec((1, 2), const),
      ],
      out_specs=[
          pl.BlockSpec((_G, H), const),
          pl.BlockSpec((_G, 1), const),
          pl.BlockSpec((_G, 2), const),
          pl.BlockSpec((1, H), const),
          pl.BlockSpec((1, H), const),
      ],
      out_shape=[
          jax.ShapeDtypeStruct((_G, H), f32),
          jax.ShapeDtypeStruct((_G, 1), f32),
          jax.ShapeDtypeStruct((_G, 2), f32),
          jax.ShapeDtypeStruct((1, H), f32),
          jax.ShapeDtypeStruct((1, H), f32),
      ],
  )

  return dict(deg=deg_kernel, acc=acc_kernel, mm1=mm1,
              stats_mm2=stats_mm2, stats_pool_head=stats_pool_head,
              RT=RT, NPAD=NPAD, EP=EP, EPP=EPP, EPPD=EPPD)


def kernel(x, edge_index, batch, W1, b1, gamma1, beta1, W2, b2, gamma2,
           beta2, fc1_W, fc1_b, fc2_W, fc2_b):
  N, F = x.shape
  H = W1.shape[1]
  E = edge_index.shape[1]
  k = _build(N, E, F, H)
  RT = k["RT"]

  NW = _NC * _NS
  EP = k["EP"]
  NPAD = k["NPAD"]
  EPP = k["EPP"]
  EPPD = k["EPPD"]
  src = edge_index[0]
  dst = edge_index[1]
  srcr = src.reshape(NW, EP)
  dstr = dst.reshape(NW, EP)
  tile = jnp.arange(NW, dtype=jnp.int32)[:, None]

  def padded(base2d, epp, trash):
    ne = epp - EP
    lane = jnp.arange(ne, dtype=jnp.int32)[None, :]
    if trash:
      p = N + (tile * ne + lane) % (NPAD - N)
    else:
      p = (tile * ne + lane) % N
    return jnp.concatenate([base2d, p], axis=1).reshape(NW * epp)

  NCHP = EPP // _CH
  sp3 = padded(srcr, EPP, False).reshape(NW, NCHP, _CH)
  dp3 = padded(dstr, EPP, True).reshape(NW, NCHP, _CH)
  epair = jnp.stack([sp3, dp3], axis=2).reshape(NW * NCHP, 2, _CH)
  dstpd = padded(dstr, EPPD, True).reshape(NW, EPPD // _CH, _CH)
  f32 = jnp.float32
  zrows = jnp.zeros((RT, F), f32)
  ones_ch = jnp.ones((_CH, 128), f32)
  zeros_rt = jnp.zeros((RT, 128), f32)

  degp = k["deg"](dstpd, ones_ch, zeros_rt)
  hp1, dinv = k["mm1"](x, W1, degp)

  accp1 = k["acc"](hp1, epair, zrows)
  hp2, _, _ = k["stats_mm2"](accp1, hp1, dinv, b1.reshape(1, H),
                             gamma1.reshape(1, H), beta1.reshape(1, H), W2)

  accp2 = k["acc"](hp2, epair, zrows)
  ps, pc, out, _, _ = k["stats_pool_head"](
      accp2, hp2, dinv, b2.reshape(1, H), gamma2.reshape(1, H),
      beta2.reshape(1, H), batch.reshape(N, 1), fc1_W, fc1_b.reshape(1, -1),
      fc2_W, fc2_b.reshape(1, -1))
  return out
